# baseline (speedup 1.0000x reference)
import functools

import jax
import jax.numpy as jnp
from jax import lax
from jax.experimental import pallas as pl
from jax.experimental.pallas import tpu as pltpu
from jax.experimental.pallas import tpu_sc as plsc

B = 16384
TEAM_DIM = 64
CITY_DIM = 32
HIDDEN = 128

NC, NS = 2, 16
NW = NC * NS
CHUNK = 128


N_TEAMS = 100000
N_FULL_BLOCKS = N_TEAMS // CHUNK
TAIL_IDS = N_TEAMS - N_FULL_BLOCKS * CHUNK
MAIN_T = (N_FULL_BLOCKS + 1) // NW
REM_FULL = N_FULL_BLOCKS - MAIN_T * NW


def _tr_body(tt_hbm, out_hbm, xb, ob, x32, o16):
    wid = lax.axis_index("s") * NC + lax.axis_index("c")
    rbase = [lax.broadcasted_iota(jnp.int32, (16,), 0) + 16 * j for j in range(4)]

    def transpose_into(i, src, dst):
        for j in range(8):
            col = jnp.zeros((16,), jnp.int32) + (2 * i + (1 if j >= 4 else 0))
            v = plsc.load_gather(src, [rbase[j % 4], col])
            dst[i, pl.ds(16 * j, 16)] = v

    def do_block(m):
        pltpu.sync_copy(tt_hbm.at[:, pl.ds(m * CHUNK, CHUNK)], xb)
        lax.fori_loop(0, 64, lambda i, c: (transpose_into(i, xb, ob), c)[1], 0)
        pltpu.sync_copy(ob, out_hbm.at[pl.ds(m * 64, 64)])

    lax.fori_loop(0, MAIN_T, lambda t, c: (do_block(wid + NW * t), c)[1], 0,
                  unroll=False)

    @pl.when(wid < REM_FULL)
    def _():
        do_block(MAIN_T * NW + wid)

    @pl.when(wid == REM_FULL)
    def _():
        base = N_FULL_BLOCKS * CHUNK
        pltpu.sync_copy(tt_hbm.at[:, pl.ds(base, TAIL_IDS)], x32)
        lax.fori_loop(0, TAIL_IDS // 2,
                      lambda i, c: (transpose_into(i, x32, o16), c)[1], 0)
        pltpu.sync_copy(o16, out_hbm.at[pl.ds(N_FULL_BLOCKS * 64, TAIL_IDS // 2)])


@functools.cache
def _make_transpose():
  return pl.kernel(
    _tr_body,
    out_type=jax.ShapeDtypeStruct((N_TEAMS // 2, 128), jnp.float32),
    mesh=plsc.VectorSubcoreMesh(core_axis_name="c", subcore_axis_name="s",
                                num_cores=NC, num_subcores=NS),
    scratch_types=[
        pltpu.VMEM((64, CHUNK), jnp.float32),
        pltpu.VMEM((64, CHUNK), jnp.float32),
        pltpu.VMEM((64, TAIL_IDS), jnp.float32),
        pltpu.VMEM((TAIL_IDS // 2, CHUNK), jnp.float32),
    ],
    compiler_params=pltpu.CompilerParams(use_tc_tiling_on_sc=True,
                                         needs_layout_passes=False),
  )


@functools.cache
def _make_gather(nrows):
  rpw = nrows // NW
  nch = rpw // CHUNK

  def _gather_body(ta_hbm, tb_hbm, tc_hbm, team_hbm, city_hbm,
                   xt_hbm, xc_hbm,
                   idx_a, idx_b, idx_c, rows_a, rows_b, rows_c, sem):
    wid = lax.axis_index("s") * NC + lax.axis_index("c")
    base = wid * rpw
    pltpu.sync_copy(ta_hbm.at[pl.ds(base, rpw)], idx_a)
    pltpu.sync_copy(tb_hbm.at[pl.ds(base, rpw)], idx_b)
    pltpu.sync_copy(tc_hbm.at[pl.ds(base, rpw)], idx_c)
    copies = []
    for j in range(nch):
        sl = pl.ds(j * CHUNK, CHUNK)
        copies.append(pltpu.async_copy(
            team_hbm.at[idx_a.at[sl]], rows_a.at[sl], sem))
        copies.append(pltpu.async_copy(
            team_hbm.at[idx_b.at[sl]], rows_b.at[sl], sem))
        copies.append(pltpu.async_copy(
            city_hbm.at[idx_c.at[sl]], rows_c.at[sl], sem))
    for c in copies:
        c.wait()
    pltpu.sync_copy(rows_a, xt_hbm.at[wid, :, pl.ds(0, TEAM_DIM)])
    pltpu.sync_copy(rows_b, xt_hbm.at[wid, :, pl.ds(TEAM_DIM, TEAM_DIM)])
    pltpu.sync_copy(rows_c, xc_hbm.at[wid, :, pl.ds(0, CITY_DIM)])

  return pl.kernel(
    _gather_body,
    out_type=(
        jax.ShapeDtypeStruct((NW, rpw, 2 * TEAM_DIM), jnp.float32),
        jax.ShapeDtypeStruct((NW, rpw, 128), jnp.float32),
    ),
    mesh=plsc.VectorSubcoreMesh(core_axis_name="c", subcore_axis_name="s",
                                num_cores=NC, num_subcores=NS),
    scratch_types=[
        pltpu.VMEM((rpw,), jnp.int32),
        pltpu.VMEM((rpw,), jnp.int32),
        pltpu.VMEM((rpw,), jnp.int32),
        pltpu.VMEM((rpw, TEAM_DIM), jnp.float32),
        pltpu.VMEM((rpw, TEAM_DIM), jnp.float32),
        pltpu.VMEM((rpw, CITY_DIM), jnp.float32),
        pltpu.SemaphoreType.DMA,
    ],
    compiler_params=pltpu.CompilerParams(use_tc_tiling_on_sc=False),
  )


def _mlp_body(xt_ref, xc_ref, xn_ref, wt_ref, wc_ref, w1n_ref,
              b1_ref, w2_ref, b2_ref, out_ref):
    blk = xt_ref.shape[0]
    lane = lax.broadcasted_iota(jnp.int32, (blk, 128), 1)
    xc = jnp.where(lane < CITY_DIM, xc_ref[...], 0.0)
    xn = xn_ref[...].T
    h = jnp.dot(xt_ref[...], wt_ref[...], preferred_element_type=jnp.float32)
    h = h + jnp.dot(xc, wc_ref[...], preferred_element_type=jnp.float32)
    h = h + jnp.dot(xn, w1n_ref[...], preferred_element_type=jnp.float32)
    h = jnp.maximum(h + b1_ref[...], 0.0)
    out_ref[...] = jnp.sum(h * w2_ref[...], axis=1) + b2_ref[0]


def _mlp(xt, xc, x_num_t, wt, wc, w1n, b1r, w2r, b2, block=4096):
    nrows = xt.shape[0]
    grid = (nrows // block,)
    wspec = lambda shape: pl.BlockSpec(shape, lambda i: (0, 0))
    return pl.pallas_call(
        _mlp_body,
        grid=grid,
        in_specs=[
            pl.BlockSpec((block, 2 * TEAM_DIM), lambda i: (i, 0)),
            pl.BlockSpec((block, 128), lambda i: (i, 0)),
            pl.BlockSpec((2, block), lambda i: (0, i)),
            wspec((2 * TEAM_DIM, HIDDEN)),
            wspec((128, HIDDEN)),
            wspec((2, HIDDEN)),
            wspec((1, HIDDEN)),
            wspec((1, HIDDEN)),
            pl.BlockSpec(memory_space=pltpu.SMEM),
        ],
        out_specs=pl.BlockSpec((block,), lambda i: (i,)),
        out_shape=jax.ShapeDtypeStruct((nrows,), jnp.float32),
    )(xt, xc, x_num_t, wt, wc, w1n, b1r, w2r, b2)


def kernel(team_a_id, team_b_id, city_id, x_num, team_emb, city_emb, W1, b1, W2, b2):
    ta = team_a_id.astype(jnp.int32)
    tb = team_b_id.astype(jnp.int32)
    tc = city_id.astype(jnp.int32)
    w1t = W1.T
    wt = w1t[0:2 * TEAM_DIM]
    wc = jnp.concatenate(
        [w1t[2 * TEAM_DIM:2 * TEAM_DIM + CITY_DIM],
         jnp.zeros((128 - CITY_DIM, HIDDEN), jnp.float32)], axis=0)
    w1n = w1t[2 * TEAM_DIM + CITY_DIM:]
    b1r = b1.reshape(1, HIDDEN)
    w2r = W2.reshape(1, HIDDEN)

    team_lin = _make_transpose()(team_emb.T).reshape(N_TEAMS, TEAM_DIM)
    half = B // 2
    gather = _make_gather(half)
    xnt = x_num.T
    outs = []
    for h in range(2):
        sl = slice(h * half, (h + 1) * half)
        xt, xc = gather(ta[sl], tb[sl], tc[sl], team_lin, city_emb)
        outs.append(_mlp(xt.reshape(half, 2 * TEAM_DIM), xc.reshape(half, 128),
                         xnt[:, sl], wt, wc, w1n, b1r, w2r, b2))
    return jnp.concatenate(outs)


# speedup vs baseline: 1.1012x; 1.1012x over previous
import functools

import jax
import jax.numpy as jnp
from jax import lax
from jax.experimental import pallas as pl
from jax.experimental.pallas import tpu as pltpu
from jax.experimental.pallas import tpu_sc as plsc

B = 16384
TEAM_DIM = 64
CITY_DIM = 32
HIDDEN = 128

NC, NS = 2, 16
NW = NC * NS
CHUNK = 128


N_TEAMS = 100000
N_FULL_BLOCKS = N_TEAMS // CHUNK
TAIL_IDS = N_TEAMS - N_FULL_BLOCKS * CHUNK
MAIN_T = (N_FULL_BLOCKS + 1) // NW
REM_FULL = N_FULL_BLOCKS - MAIN_T * NW


def _tr_body(tt_hbm, out_hbm, xb0, xb1, ob0, ob1, x32, o16,
             si0, si1, so0, so1):
    wid = lax.axis_index("s") * NC + lax.axis_index("c")
    rbase = [lax.broadcasted_iota(jnp.int32, (16,), 0) + 16 * j for j in range(4)]
    xbs, obs = [xb0, xb1], [ob0, ob1]
    sis, sos = [si0, si1], [so0, so1]

    def transpose_into(i, src, dst):
        for j in range(8):
            col = jnp.zeros((16,), jnp.int32) + (2 * i + (1 if j >= 4 else 0))
            v = plsc.load_gather(src, [rbase[j % 4], col])
            dst[i, pl.ds(16 * j, 16)] = v

    def compute(src, dst):
        lax.fori_loop(0, 64, lambda i, c: (transpose_into(i, src, dst), c)[1],
                      0, unroll=4)

    def fire_in(t, s):
        m = wid + NW * t
        return pltpu.async_copy(tt_hbm.at[:, pl.ds(m * CHUNK, CHUNK)],
                                xbs[s], sis[s])

    def fire_out(t, s):
        m = wid + NW * t
        return pltpu.async_copy(obs[s], out_hbm.at[pl.ds(m * 64, 64)], sos[s])

    cin = [fire_in(0, 0), fire_in(1, 1)]
    couts = [None, None]
    for t in range(MAIN_T):
        s = t % 2
        cin[s].wait()
        if couts[s] is not None:
            couts[s].wait()
        compute(xbs[s], obs[s])
        couts[s] = fire_out(t, s)
        if t + 2 < MAIN_T:
            cin[s] = fire_in(t + 2, s)
    couts[0].wait()
    couts[1].wait()

    @pl.when(wid < REM_FULL)
    def _():
        m = MAIN_T * NW + wid
        pltpu.sync_copy(tt_hbm.at[:, pl.ds(m * CHUNK, CHUNK)], xb0)
        compute(xb0, ob0)
        pltpu.sync_copy(ob0, out_hbm.at[pl.ds(m * 64, 64)])

    @pl.when(wid == REM_FULL)
    def _():
        base = N_FULL_BLOCKS * CHUNK
        pltpu.sync_copy(tt_hbm.at[:, pl.ds(base, TAIL_IDS)], x32)
        lax.fori_loop(0, TAIL_IDS // 2,
                      lambda i, c: (transpose_into(i, x32, o16), c)[1], 0)
        pltpu.sync_copy(o16, out_hbm.at[pl.ds(N_FULL_BLOCKS * 64, TAIL_IDS // 2)])


@functools.cache
def _make_transpose():
  return pl.kernel(
    _tr_body,
    out_type=jax.ShapeDtypeStruct((N_TEAMS // 2, 128), jnp.float32),
    mesh=plsc.VectorSubcoreMesh(core_axis_name="c", subcore_axis_name="s",
                                num_cores=NC, num_subcores=NS),
    scratch_types=[
        pltpu.VMEM((64, CHUNK), jnp.float32),
        pltpu.VMEM((64, CHUNK), jnp.float32),
        pltpu.VMEM((64, CHUNK), jnp.float32),
        pltpu.VMEM((64, CHUNK), jnp.float32),
        pltpu.VMEM((64, TAIL_IDS), jnp.float32),
        pltpu.VMEM((TAIL_IDS // 2, CHUNK), jnp.float32),
        pltpu.SemaphoreType.DMA,
        pltpu.SemaphoreType.DMA,
        pltpu.SemaphoreType.DMA,
        pltpu.SemaphoreType.DMA,
    ],
    compiler_params=pltpu.CompilerParams(use_tc_tiling_on_sc=True,
                                         needs_layout_passes=False),
  )


@functools.cache
def _make_gather(nrows):
  rpw = nrows // NW
  nch = rpw // CHUNK

  def _gather_body(ta_hbm, tb_hbm, tc_hbm, team_hbm, city_hbm,
                   xt_hbm, xc_hbm,
                   idx_a, idx_b, idx_c, rows_a, rows_b, rows_c, sem):
    wid = lax.axis_index("s") * NC + lax.axis_index("c")
    base = wid * rpw
    pltpu.sync_copy(ta_hbm.at[pl.ds(base, rpw)], idx_a)
    pltpu.sync_copy(tb_hbm.at[pl.ds(base, rpw)], idx_b)
    pltpu.sync_copy(tc_hbm.at[pl.ds(base, rpw)], idx_c)
    copies = []
    for j in range(nch):
        sl = pl.ds(j * CHUNK, CHUNK)
        copies.append(pltpu.async_copy(
            team_hbm.at[idx_a.at[sl]], rows_a.at[sl], sem))
        copies.append(pltpu.async_copy(
            team_hbm.at[idx_b.at[sl]], rows_b.at[sl], sem))
        copies.append(pltpu.async_copy(
            city_hbm.at[idx_c.at[sl]], rows_c.at[sl], sem))
    for c in copies:
        c.wait()
    pltpu.sync_copy(rows_a, xt_hbm.at[wid, :, pl.ds(0, TEAM_DIM)])
    pltpu.sync_copy(rows_b, xt_hbm.at[wid, :, pl.ds(TEAM_DIM, TEAM_DIM)])
    pltpu.sync_copy(rows_c, xc_hbm.at[wid, :, pl.ds(0, CITY_DIM)])

  return pl.kernel(
    _gather_body,
    out_type=(
        jax.ShapeDtypeStruct((NW, rpw, 2 * TEAM_DIM), jnp.float32),
        jax.ShapeDtypeStruct((NW, rpw, 128), jnp.float32),
    ),
    mesh=plsc.VectorSubcoreMesh(core_axis_name="c", subcore_axis_name="s",
                                num_cores=NC, num_subcores=NS),
    scratch_types=[
        pltpu.VMEM((rpw,), jnp.int32),
        pltpu.VMEM((rpw,), jnp.int32),
        pltpu.VMEM((rpw,), jnp.int32),
        pltpu.VMEM((rpw, TEAM_DIM), jnp.float32),
        pltpu.VMEM((rpw, TEAM_DIM), jnp.float32),
        pltpu.VMEM((rpw, CITY_DIM), jnp.float32),
        pltpu.SemaphoreType.DMA,
    ],
    compiler_params=pltpu.CompilerParams(use_tc_tiling_on_sc=False),
  )


def _mlp_body(xt_ref, xc_ref, xn_ref, wt_ref, wc_ref, w1n_ref,
              b1_ref, w2_ref, b2_ref, out_ref):
    blk = xt_ref.shape[0]
    lane = lax.broadcasted_iota(jnp.int32, (blk, 128), 1)
    xc = jnp.where(lane < CITY_DIM, xc_ref[...], 0.0)
    xn = xn_ref[...].T
    h = jnp.dot(xt_ref[...], wt_ref[...], preferred_element_type=jnp.float32)
    h = h + jnp.dot(xc, wc_ref[...], preferred_element_type=jnp.float32)
    h = h + jnp.dot(xn, w1n_ref[...], preferred_element_type=jnp.float32)
    h = jnp.maximum(h + b1_ref[...], 0.0)
    out_ref[...] = jnp.sum(h * w2_ref[...], axis=1) + b2_ref[0]


def _mlp(xt, xc, x_num_t, wt, wc, w1n, b1r, w2r, b2, block=4096):
    nrows = xt.shape[0]
    grid = (nrows // block,)
    wspec = lambda shape: pl.BlockSpec(shape, lambda i: (0, 0))
    return pl.pallas_call(
        _mlp_body,
        grid=grid,
        in_specs=[
            pl.BlockSpec((block, 2 * TEAM_DIM), lambda i: (i, 0)),
            pl.BlockSpec((block, 128), lambda i: (i, 0)),
            pl.BlockSpec((2, block), lambda i: (0, i)),
            wspec((2 * TEAM_DIM, HIDDEN)),
            wspec((128, HIDDEN)),
            wspec((2, HIDDEN)),
            wspec((1, HIDDEN)),
            wspec((1, HIDDEN)),
            pl.BlockSpec(memory_space=pltpu.SMEM),
        ],
        out_specs=pl.BlockSpec((block,), lambda i: (i,)),
        out_shape=jax.ShapeDtypeStruct((nrows,), jnp.float32),
    )(xt, xc, x_num_t, wt, wc, w1n, b1r, w2r, b2)


def kernel(team_a_id, team_b_id, city_id, x_num, team_emb, city_emb, W1, b1, W2, b2):
    ta = team_a_id.astype(jnp.int32)
    tb = team_b_id.astype(jnp.int32)
    tc = city_id.astype(jnp.int32)
    w1t = W1.T
    wt = w1t[0:2 * TEAM_DIM]
    wc = jnp.concatenate(
        [w1t[2 * TEAM_DIM:2 * TEAM_DIM + CITY_DIM],
         jnp.zeros((128 - CITY_DIM, HIDDEN), jnp.float32)], axis=0)
    w1n = w1t[2 * TEAM_DIM + CITY_DIM:]
    b1r = b1.reshape(1, HIDDEN)
    w2r = W2.reshape(1, HIDDEN)

    team_lin = _make_transpose()(team_emb.T).reshape(N_TEAMS, TEAM_DIM)
    half = B // 2
    gather = _make_gather(half)
    xnt = x_num.T
    outs = []
    for h in range(2):
        sl = slice(h * half, (h + 1) * half)
        xt, xc = gather(ta[sl], tb[sl], tc[sl], team_lin, city_emb)
        outs.append(_mlp(xt.reshape(half, 2 * TEAM_DIM), xc.reshape(half, 128),
                         xnt[:, sl], wt, wc, w1n, b1r, w2r, b2))
    return jnp.concatenate(outs)


# speedup vs baseline: 1.1035x; 1.0021x over previous
import functools

import jax
import jax.numpy as jnp
from jax import lax
from jax.experimental import pallas as pl
from jax.experimental.pallas import tpu as pltpu
from jax.experimental.pallas import tpu_sc as plsc

B = 16384
TEAM_DIM = 64
CITY_DIM = 32
HIDDEN = 128

NC, NS = 2, 16
NW = NC * NS
CHUNK = 128


N_TEAMS = 100000
N_FULL_BLOCKS = N_TEAMS // CHUNK
TAIL_IDS = N_TEAMS - N_FULL_BLOCKS * CHUNK
MAIN_T = (N_FULL_BLOCKS + 1) // NW
REM_FULL = N_FULL_BLOCKS - MAIN_T * NW


def _tr_body(tt_hbm, out_hbm, xb0, xb1, ob0, ob1, x32, o16,
             si0, si1, so0, so1):
    wid = lax.axis_index("s") * NC + lax.axis_index("c")
    rbase = [lax.broadcasted_iota(jnp.int32, (16,), 0) + 16 * j for j in range(4)]
    xbs, obs = [xb0, xb1], [ob0, ob1]
    sis, sos = [si0, si1], [so0, so1]

    def transpose_into(i, src, dst):
        for j in range(8):
            col = jnp.zeros((16,), jnp.int32) + (2 * i + (1 if j >= 4 else 0))
            v = plsc.load_gather(src, [rbase[j % 4], col])
            dst[i, pl.ds(16 * j, 16)] = v

    def compute(src, dst):
        lax.fori_loop(0, 64, lambda i, c: (transpose_into(i, src, dst), c)[1],
                      0, unroll=4)

    def fire_in(t, s):
        m = wid + NW * t
        return pltpu.async_copy(tt_hbm.at[:, pl.ds(m * CHUNK, CHUNK)],
                                xbs[s].at[:, pl.ds(0, CHUNK)], sis[s])

    def fire_out(t, s):
        m = wid + NW * t
        return pltpu.async_copy(obs[s], out_hbm.at[pl.ds(m * 64, 64)], sos[s])

    cin = [fire_in(0, 0), fire_in(1, 1)]
    couts = [None, None]
    for t in range(MAIN_T):
        s = t % 2
        cin[s].wait()
        if couts[s] is not None:
            couts[s].wait()
        compute(xbs[s], obs[s])
        couts[s] = fire_out(t, s)
        if t + 2 < MAIN_T:
            cin[s] = fire_in(t + 2, s)
    couts[0].wait()
    couts[1].wait()

    @pl.when(wid < REM_FULL)
    def _():
        m = MAIN_T * NW + wid
        pltpu.sync_copy(tt_hbm.at[:, pl.ds(m * CHUNK, CHUNK)],
                        xb0.at[:, pl.ds(0, CHUNK)])
        compute(xb0, ob0)
        pltpu.sync_copy(ob0, out_hbm.at[pl.ds(m * 64, 64)])

    @pl.when(wid == REM_FULL)
    def _():
        base = N_FULL_BLOCKS * CHUNK
        pltpu.sync_copy(tt_hbm.at[:, pl.ds(base, TAIL_IDS)], x32)
        lax.fori_loop(0, TAIL_IDS // 2,
                      lambda i, c: (transpose_into(i, x32, o16), c)[1], 0)
        pltpu.sync_copy(o16, out_hbm.at[pl.ds(N_FULL_BLOCKS * 64, TAIL_IDS // 2)])


@functools.cache
def _make_transpose():
  return pl.kernel(
    _tr_body,
    out_type=jax.ShapeDtypeStruct((N_TEAMS // 2, 128), jnp.float32),
    mesh=plsc.VectorSubcoreMesh(core_axis_name="c", subcore_axis_name="s",
                                num_cores=NC, num_subcores=NS),
    scratch_types=[
        pltpu.VMEM((64, CHUNK + 1), jnp.float32),
        pltpu.VMEM((64, CHUNK + 1), jnp.float32),
        pltpu.VMEM((64, CHUNK), jnp.float32),
        pltpu.VMEM((64, CHUNK), jnp.float32),
        pltpu.VMEM((64, TAIL_IDS), jnp.float32),
        pltpu.VMEM((TAIL_IDS // 2, CHUNK), jnp.float32),
        pltpu.SemaphoreType.DMA,
        pltpu.SemaphoreType.DMA,
        pltpu.SemaphoreType.DMA,
        pltpu.SemaphoreType.DMA,
    ],
    compiler_params=pltpu.CompilerParams(use_tc_tiling_on_sc=True,
                                         needs_layout_passes=False),
  )


@functools.cache
def _make_gather(nrows):
  rpw = nrows // NW
  nch = rpw // CHUNK

  def _gather_body(ta_hbm, tb_hbm, tc_hbm, team_hbm, city_hbm,
                   xt_hbm, xc_hbm,
                   idx_a, idx_b, idx_c, rows_a, rows_b, rows_c, sem):
    wid = lax.axis_index("s") * NC + lax.axis_index("c")
    base = wid * rpw
    pltpu.sync_copy(ta_hbm.at[pl.ds(base, rpw)], idx_a)
    pltpu.sync_copy(tb_hbm.at[pl.ds(base, rpw)], idx_b)
    pltpu.sync_copy(tc_hbm.at[pl.ds(base, rpw)], idx_c)
    copies = []
    for j in range(nch):
        sl = pl.ds(j * CHUNK, CHUNK)
        copies.append(pltpu.async_copy(
            team_hbm.at[idx_a.at[sl]], rows_a.at[sl], sem))
        copies.append(pltpu.async_copy(
            team_hbm.at[idx_b.at[sl]], rows_b.at[sl], sem))
        copies.append(pltpu.async_copy(
            city_hbm.at[idx_c.at[sl]], rows_c.at[sl], sem))
    for c in copies:
        c.wait()
    pltpu.sync_copy(rows_a, xt_hbm.at[wid, :, pl.ds(0, TEAM_DIM)])
    pltpu.sync_copy(rows_b, xt_hbm.at[wid, :, pl.ds(TEAM_DIM, TEAM_DIM)])
    pltpu.sync_copy(rows_c, xc_hbm.at[wid, :, pl.ds(0, CITY_DIM)])

  return pl.kernel(
    _gather_body,
    out_type=(
        jax.ShapeDtypeStruct((NW, rpw, 2 * TEAM_DIM), jnp.float32),
        jax.ShapeDtypeStruct((NW, rpw, 128), jnp.float32),
    ),
    mesh=plsc.VectorSubcoreMesh(core_axis_name="c", subcore_axis_name="s",
                                num_cores=NC, num_subcores=NS),
    scratch_types=[
        pltpu.VMEM((rpw,), jnp.int32),
        pltpu.VMEM((rpw,), jnp.int32),
        pltpu.VMEM((rpw,), jnp.int32),
        pltpu.VMEM((rpw, TEAM_DIM), jnp.float32),
        pltpu.VMEM((rpw, TEAM_DIM), jnp.float32),
        pltpu.VMEM((rpw, CITY_DIM), jnp.float32),
        pltpu.SemaphoreType.DMA,
    ],
    compiler_params=pltpu.CompilerParams(use_tc_tiling_on_sc=False),
  )


def _mlp_body(xt_ref, xc_ref, xn_ref, wt_ref, wc_ref, w1n_ref,
              b1_ref, w2_ref, b2_ref, out_ref):
    blk = xt_ref.shape[0]
    lane = lax.broadcasted_iota(jnp.int32, (blk, 128), 1)
    xc = jnp.where(lane < CITY_DIM, xc_ref[...], 0.0)
    xn = xn_ref[...].T
    h = jnp.dot(xt_ref[...], wt_ref[...], preferred_element_type=jnp.float32)
    h = h + jnp.dot(xc, wc_ref[...], preferred_element_type=jnp.float32)
    h = h + jnp.dot(xn, w1n_ref[...], preferred_element_type=jnp.float32)
    h = jnp.maximum(h + b1_ref[...], 0.0)
    out_ref[...] = jnp.sum(h * w2_ref[...], axis=1) + b2_ref[0]


def _mlp(xt, xc, x_num_t, wt, wc, w1n, b1r, w2r, b2, block=4096):
    nrows = xt.shape[0]
    grid = (nrows // block,)
    wspec = lambda shape: pl.BlockSpec(shape, lambda i: (0, 0))
    return pl.pallas_call(
        _mlp_body,
        grid=grid,
        in_specs=[
            pl.BlockSpec((block, 2 * TEAM_DIM), lambda i: (i, 0)),
            pl.BlockSpec((block, 128), lambda i: (i, 0)),
            pl.BlockSpec((2, block), lambda i: (0, i)),
            wspec((2 * TEAM_DIM, HIDDEN)),
            wspec((128, HIDDEN)),
            wspec((2, HIDDEN)),
            wspec((1, HIDDEN)),
            wspec((1, HIDDEN)),
            pl.BlockSpec(memory_space=pltpu.SMEM),
        ],
        out_specs=pl.BlockSpec((block,), lambda i: (i,)),
        out_shape=jax.ShapeDtypeStruct((nrows,), jnp.float32),
    )(xt, xc, x_num_t, wt, wc, w1n, b1r, w2r, b2)


def kernel(team_a_id, team_b_id, city_id, x_num, team_emb, city_emb, W1, b1, W2, b2):
    ta = team_a_id.astype(jnp.int32)
    tb = team_b_id.astype(jnp.int32)
    tc = city_id.astype(jnp.int32)
    w1t = W1.T
    wt = w1t[0:2 * TEAM_DIM]
    wc = jnp.concatenate(
        [w1t[2 * TEAM_DIM:2 * TEAM_DIM + CITY_DIM],
         jnp.zeros((128 - CITY_DIM, HIDDEN), jnp.float32)], axis=0)
    w1n = w1t[2 * TEAM_DIM + CITY_DIM:]
    b1r = b1.reshape(1, HIDDEN)
    w2r = W2.reshape(1, HIDDEN)

    team_lin = _make_transpose()(team_emb.T).reshape(N_TEAMS, TEAM_DIM)
    half = B // 2
    gather = _make_gather(half)
    xnt = x_num.T
    outs = []
    for h in range(2):
        sl = slice(h * half, (h + 1) * half)
        xt, xc = gather(ta[sl], tb[sl], tc[sl], team_lin, city_emb)
        outs.append(_mlp(xt.reshape(half, 2 * TEAM_DIM), xc.reshape(half, 128),
                         xnt[:, sl], wt, wc, w1n, b1r, w2r, b2))
    return jnp.concatenate(outs)


# speedup vs baseline: 1.6744x; 1.5173x over previous
import functools

import jax
import jax.numpy as jnp
from jax import lax
from jax.experimental import pallas as pl
from jax.experimental.pallas import tpu as pltpu
from jax.experimental.pallas import tpu_sc as plsc

B = 16384
TEAM_DIM = 64
CITY_DIM = 32
HIDDEN = 128

NC, NS = 2, 16
NW = NC * NS
CHUNK = 128


N_TEAMS = 100000
N_FULL_BLOCKS = N_TEAMS // CHUNK
TAIL_IDS = N_TEAMS - N_FULL_BLOCKS * CHUNK
MAIN_T = (N_FULL_BLOCKS + 1) // NW
REM_FULL = N_FULL_BLOCKS - MAIN_T * NW


def _tr_body(tt_hbm, out_hbm, xb0, xb1, ob0, ob1, x32, o16,
             si0, si1, so0, so1):
    wid = lax.axis_index("s") * NC + lax.axis_index("c")
    rbase = [lax.broadcasted_iota(jnp.int32, (16,), 0) + 16 * j for j in range(4)]
    xbs, obs = [xb0, xb1], [ob0, ob1]
    sis, sos = [si0, si1], [so0, so1]

    def transpose_into(i, src, dst):
        for j in range(8):
            col = jnp.zeros((16,), jnp.int32) + (2 * i + (1 if j >= 4 else 0))
            v = plsc.load_gather(src, [rbase[j % 4], col])
            dst[i, pl.ds(16 * j, 16)] = v

    def compute(src, dst):
        @plsc.parallel_loop(0, 64, unroll=4)
        def _(i):
            transpose_into(i, src, dst)

    def fire_in(t, s):
        m = wid + NW * t
        return pltpu.async_copy(tt_hbm.at[:, pl.ds(m * CHUNK, CHUNK)],
                                xbs[s].at[:, pl.ds(0, CHUNK)], sis[s])

    def fire_out(t, s):
        m = wid + NW * t
        return pltpu.async_copy(obs[s], out_hbm.at[pl.ds(m * 64, 64)], sos[s])

    cin = [fire_in(0, 0), fire_in(1, 1)]
    couts = [None, None]
    for t in range(MAIN_T):
        s = t % 2
        cin[s].wait()
        if couts[s] is not None:
            couts[s].wait()
        compute(xbs[s], obs[s])
        couts[s] = fire_out(t, s)
        if t + 2 < MAIN_T:
            cin[s] = fire_in(t + 2, s)
    couts[0].wait()
    couts[1].wait()

    @pl.when(wid < REM_FULL)
    def _():
        m = MAIN_T * NW + wid
        pltpu.sync_copy(tt_hbm.at[:, pl.ds(m * CHUNK, CHUNK)],
                        xb0.at[:, pl.ds(0, CHUNK)])
        compute(xb0, ob0)
        pltpu.sync_copy(ob0, out_hbm.at[pl.ds(m * 64, 64)])

    @pl.when(wid == REM_FULL)
    def _():
        base = N_FULL_BLOCKS * CHUNK
        pltpu.sync_copy(tt_hbm.at[:, pl.ds(base, TAIL_IDS)], x32)
        lax.fori_loop(0, TAIL_IDS // 2,
                      lambda i, c: (transpose_into(i, x32, o16), c)[1], 0)
        pltpu.sync_copy(o16, out_hbm.at[pl.ds(N_FULL_BLOCKS * 64, TAIL_IDS // 2)])


@functools.cache
def _make_transpose():
  return pl.kernel(
    _tr_body,
    out_type=jax.ShapeDtypeStruct((N_TEAMS // 2, 128), jnp.float32),
    mesh=plsc.VectorSubcoreMesh(core_axis_name="c", subcore_axis_name="s",
                                num_cores=NC, num_subcores=NS),
    scratch_types=[
        pltpu.VMEM((64, CHUNK + 1), jnp.float32),
        pltpu.VMEM((64, CHUNK + 1), jnp.float32),
        pltpu.VMEM((64, CHUNK), jnp.float32),
        pltpu.VMEM((64, CHUNK), jnp.float32),
        pltpu.VMEM((64, TAIL_IDS), jnp.float32),
        pltpu.VMEM((TAIL_IDS // 2, CHUNK), jnp.float32),
        pltpu.SemaphoreType.DMA,
        pltpu.SemaphoreType.DMA,
        pltpu.SemaphoreType.DMA,
        pltpu.SemaphoreType.DMA,
    ],
    compiler_params=pltpu.CompilerParams(use_tc_tiling_on_sc=True,
                                         needs_layout_passes=False),
  )


@functools.cache
def _make_gather(nrows):
  rpw = nrows // NW
  nch = rpw // CHUNK

  def _gather_body(ta_hbm, tb_hbm, tc_hbm, team_hbm, city_hbm,
                   xt_hbm, xc_hbm,
                   idx_a, idx_b, idx_c, rows_a, rows_b, rows_c, sem):
    wid = lax.axis_index("s") * NC + lax.axis_index("c")
    base = wid * rpw
    pltpu.sync_copy(ta_hbm.at[pl.ds(base, rpw)], idx_a)
    pltpu.sync_copy(tb_hbm.at[pl.ds(base, rpw)], idx_b)
    pltpu.sync_copy(tc_hbm.at[pl.ds(base, rpw)], idx_c)
    copies = []
    for j in range(nch):
        sl = pl.ds(j * CHUNK, CHUNK)
        copies.append(pltpu.async_copy(
            team_hbm.at[idx_a.at[sl]], rows_a.at[sl], sem))
        copies.append(pltpu.async_copy(
            team_hbm.at[idx_b.at[sl]], rows_b.at[sl], sem))
        copies.append(pltpu.async_copy(
            city_hbm.at[idx_c.at[sl]], rows_c.at[sl], sem))
    for c in copies:
        c.wait()
    pltpu.sync_copy(rows_a, xt_hbm.at[wid, :, pl.ds(0, TEAM_DIM)])
    pltpu.sync_copy(rows_b, xt_hbm.at[wid, :, pl.ds(TEAM_DIM, TEAM_DIM)])
    pltpu.sync_copy(rows_c, xc_hbm.at[wid, :, pl.ds(0, CITY_DIM)])

  return pl.kernel(
    _gather_body,
    out_type=(
        jax.ShapeDtypeStruct((NW, rpw, 2 * TEAM_DIM), jnp.float32),
        jax.ShapeDtypeStruct((NW, rpw, 128), jnp.float32),
    ),
    mesh=plsc.VectorSubcoreMesh(core_axis_name="c", subcore_axis_name="s",
                                num_cores=NC, num_subcores=NS),
    scratch_types=[
        pltpu.VMEM((rpw,), jnp.int32),
        pltpu.VMEM((rpw,), jnp.int32),
        pltpu.VMEM((rpw,), jnp.int32),
        pltpu.VMEM((rpw, TEAM_DIM), jnp.float32),
        pltpu.VMEM((rpw, TEAM_DIM), jnp.float32),
        pltpu.VMEM((rpw, CITY_DIM), jnp.float32),
        pltpu.SemaphoreType.DMA,
    ],
    compiler_params=pltpu.CompilerParams(use_tc_tiling_on_sc=False),
  )


def _mlp_body(xt_ref, xc_ref, xn_ref, wt_ref, wc_ref, w1n_ref,
              b1_ref, w2_ref, b2_ref, out_ref):
    blk = xt_ref.shape[0]
    lane = lax.broadcasted_iota(jnp.int32, (blk, 128), 1)
    xc = jnp.where(lane < CITY_DIM, xc_ref[...], 0.0)
    xn = xn_ref[...].T
    h = jnp.dot(xt_ref[...], wt_ref[...], preferred_element_type=jnp.float32)
    h = h + jnp.dot(xc, wc_ref[...], preferred_element_type=jnp.float32)
    h = h + jnp.dot(xn, w1n_ref[...], preferred_element_type=jnp.float32)
    h = jnp.maximum(h + b1_ref[...], 0.0)
    out_ref[...] = jnp.sum(h * w2_ref[...], axis=1) + b2_ref[0]


def _mlp(xt, xc, x_num_t, wt, wc, w1n, b1r, w2r, b2, block=4096):
    nrows = xt.shape[0]
    grid = (nrows // block,)
    wspec = lambda shape: pl.BlockSpec(shape, lambda i: (0, 0))
    return pl.pallas_call(
        _mlp_body,
        grid=grid,
        in_specs=[
            pl.BlockSpec((block, 2 * TEAM_DIM), lambda i: (i, 0)),
            pl.BlockSpec((block, 128), lambda i: (i, 0)),
            pl.BlockSpec((2, block), lambda i: (0, i)),
            wspec((2 * TEAM_DIM, HIDDEN)),
            wspec((128, HIDDEN)),
            wspec((2, HIDDEN)),
            wspec((1, HIDDEN)),
            wspec((1, HIDDEN)),
            pl.BlockSpec(memory_space=pltpu.SMEM),
        ],
        out_specs=pl.BlockSpec((block,), lambda i: (i,)),
        out_shape=jax.ShapeDtypeStruct((nrows,), jnp.float32),
    )(xt, xc, x_num_t, wt, wc, w1n, b1r, w2r, b2)


def kernel(team_a_id, team_b_id, city_id, x_num, team_emb, city_emb, W1, b1, W2, b2):
    ta = team_a_id.astype(jnp.int32)
    tb = team_b_id.astype(jnp.int32)
    tc = city_id.astype(jnp.int32)
    w1t = W1.T
    wt = w1t[0:2 * TEAM_DIM]
    wc = jnp.concatenate(
        [w1t[2 * TEAM_DIM:2 * TEAM_DIM + CITY_DIM],
         jnp.zeros((128 - CITY_DIM, HIDDEN), jnp.float32)], axis=0)
    w1n = w1t[2 * TEAM_DIM + CITY_DIM:]
    b1r = b1.reshape(1, HIDDEN)
    w2r = W2.reshape(1, HIDDEN)

    team_lin = _make_transpose()(team_emb.T).reshape(N_TEAMS, TEAM_DIM)
    half = B // 2
    gather = _make_gather(half)
    xnt = x_num.T
    outs = []
    for h in range(2):
        sl = slice(h * half, (h + 1) * half)
        xt, xc = gather(ta[sl], tb[sl], tc[sl], team_lin, city_emb)
        outs.append(_mlp(xt.reshape(half, 2 * TEAM_DIM), xc.reshape(half, 128),
                         xnt[:, sl], wt, wc, w1n, b1r, w2r, b2))
    return jnp.concatenate(outs)


# speedup vs baseline: 1.6841x; 1.0058x over previous
import functools

import jax
import jax.numpy as jnp
from jax import lax
from jax.experimental import pallas as pl
from jax.experimental.pallas import tpu as pltpu
from jax.experimental.pallas import tpu_sc as plsc

B = 16384
TEAM_DIM = 64
CITY_DIM = 32
HIDDEN = 128

NC, NS = 2, 16
NW = NC * NS
CHUNK = 128


N_TEAMS = 100000
N_FULL_BLOCKS = N_TEAMS // CHUNK
TAIL_IDS = N_TEAMS - N_FULL_BLOCKS * CHUNK
MAIN_T = (N_FULL_BLOCKS + 1) // NW
REM_FULL = N_FULL_BLOCKS - MAIN_T * NW


def _tr_body(tt_hbm, out_hbm, xb0, xb1, ob0, ob1, x32, o16,
             si0, si1, so0, so1):
    wid = lax.axis_index("s") * NC + lax.axis_index("c")
    rbase = [lax.broadcasted_iota(jnp.int32, (16,), 0) + 16 * j for j in range(4)]
    xbs, obs = [xb0, xb1], [ob0, ob1]
    sis, sos = [si0, si1], [so0, so1]

    def transpose_into(i, src, dst):
        cols = [jnp.zeros((16,), jnp.int32) + 2 * i]
        cols.append(cols[0] + 1)
        for j in range(8):
            v = plsc.load_gather(src, [rbase[j % 4], cols[j // 4]])
            dst[i, pl.ds(16 * j, 16)] = v

    def compute(src, dst):
        @plsc.parallel_loop(0, 64, unroll=8)
        def _(i):
            transpose_into(i, src, dst)

    def fire_in(t, s):
        m = wid + NW * t
        return pltpu.async_copy(tt_hbm.at[:, pl.ds(m * CHUNK, CHUNK)],
                                xbs[s].at[:, pl.ds(0, CHUNK)], sis[s])

    def fire_out(t, s):
        m = wid + NW * t
        return pltpu.async_copy(obs[s], out_hbm.at[pl.ds(m * 64, 64)], sos[s])

    cin = [fire_in(0, 0), fire_in(1, 1)]
    couts = [None, None]
    for t in range(MAIN_T):
        s = t % 2
        cin[s].wait()
        if couts[s] is not None:
            couts[s].wait()
        compute(xbs[s], obs[s])
        couts[s] = fire_out(t, s)
        if t + 2 < MAIN_T:
            cin[s] = fire_in(t + 2, s)
    couts[0].wait()
    couts[1].wait()

    @pl.when(wid < REM_FULL)
    def _():
        m = MAIN_T * NW + wid
        pltpu.sync_copy(tt_hbm.at[:, pl.ds(m * CHUNK, CHUNK)],
                        xb0.at[:, pl.ds(0, CHUNK)])
        compute(xb0, ob0)
        pltpu.sync_copy(ob0, out_hbm.at[pl.ds(m * 64, 64)])

    @pl.when(wid == REM_FULL)
    def _():
        base = N_FULL_BLOCKS * CHUNK
        pltpu.sync_copy(tt_hbm.at[:, pl.ds(base, TAIL_IDS)], x32)
        lax.fori_loop(0, TAIL_IDS // 2,
                      lambda i, c: (transpose_into(i, x32, o16), c)[1], 0)
        pltpu.sync_copy(o16, out_hbm.at[pl.ds(N_FULL_BLOCKS * 64, TAIL_IDS // 2)])


@functools.cache
def _make_transpose():
  return pl.kernel(
    _tr_body,
    out_type=jax.ShapeDtypeStruct((N_TEAMS // 2, 128), jnp.float32),
    mesh=plsc.VectorSubcoreMesh(core_axis_name="c", subcore_axis_name="s",
                                num_cores=NC, num_subcores=NS),
    scratch_types=[
        pltpu.VMEM((64, CHUNK + 1), jnp.float32),
        pltpu.VMEM((64, CHUNK + 1), jnp.float32),
        pltpu.VMEM((64, CHUNK), jnp.float32),
        pltpu.VMEM((64, CHUNK), jnp.float32),
        pltpu.VMEM((64, TAIL_IDS), jnp.float32),
        pltpu.VMEM((TAIL_IDS // 2, CHUNK), jnp.float32),
        pltpu.SemaphoreType.DMA,
        pltpu.SemaphoreType.DMA,
        pltpu.SemaphoreType.DMA,
        pltpu.SemaphoreType.DMA,
    ],
    compiler_params=pltpu.CompilerParams(use_tc_tiling_on_sc=True,
                                         needs_layout_passes=False),
  )


@functools.cache
def _make_gather(nrows):
  rpw = nrows // NW
  nch = rpw // CHUNK

  def _gather_body(ta_hbm, tb_hbm, tc_hbm, team_hbm, city_hbm,
                   xt_hbm, xc_hbm,
                   idx_a, idx_b, idx_c, rows_a, rows_b, rows_c, sem):
    wid = lax.axis_index("s") * NC + lax.axis_index("c")
    base = wid * rpw
    pltpu.sync_copy(ta_hbm.at[pl.ds(base, rpw)], idx_a)
    pltpu.sync_copy(tb_hbm.at[pl.ds(base, rpw)], idx_b)
    pltpu.sync_copy(tc_hbm.at[pl.ds(base, rpw)], idx_c)
    copies = []
    for j in range(nch):
        sl = pl.ds(j * CHUNK, CHUNK)
        copies.append(pltpu.async_copy(
            team_hbm.at[idx_a.at[sl]], rows_a.at[sl], sem))
        copies.append(pltpu.async_copy(
            team_hbm.at[idx_b.at[sl]], rows_b.at[sl], sem))
        copies.append(pltpu.async_copy(
            city_hbm.at[idx_c.at[sl]], rows_c.at[sl], sem))
    for c in copies:
        c.wait()
    pltpu.sync_copy(rows_a, xt_hbm.at[wid, :, pl.ds(0, TEAM_DIM)])
    pltpu.sync_copy(rows_b, xt_hbm.at[wid, :, pl.ds(TEAM_DIM, TEAM_DIM)])
    pltpu.sync_copy(rows_c, xc_hbm.at[wid, :, pl.ds(0, CITY_DIM)])

  return pl.kernel(
    _gather_body,
    out_type=(
        jax.ShapeDtypeStruct((NW, rpw, 2 * TEAM_DIM), jnp.float32),
        jax.ShapeDtypeStruct((NW, rpw, 128), jnp.float32),
    ),
    mesh=plsc.VectorSubcoreMesh(core_axis_name="c", subcore_axis_name="s",
                                num_cores=NC, num_subcores=NS),
    scratch_types=[
        pltpu.VMEM((rpw,), jnp.int32),
        pltpu.VMEM((rpw,), jnp.int32),
        pltpu.VMEM((rpw,), jnp.int32),
        pltpu.VMEM((rpw, TEAM_DIM), jnp.float32),
        pltpu.VMEM((rpw, TEAM_DIM), jnp.float32),
        pltpu.VMEM((rpw, CITY_DIM), jnp.float32),
        pltpu.SemaphoreType.DMA,
    ],
    compiler_params=pltpu.CompilerParams(use_tc_tiling_on_sc=False),
  )


def _mlp_body(xt_ref, xc_ref, xn_ref, wt_ref, wc_ref, w1n_ref,
              b1_ref, w2_ref, b2_ref, out_ref):
    blk = xt_ref.shape[0]
    lane = lax.broadcasted_iota(jnp.int32, (blk, 128), 1)
    xc = jnp.where(lane < CITY_DIM, xc_ref[...], 0.0)
    xn = xn_ref[...].T
    h = jnp.dot(xt_ref[...], wt_ref[...], preferred_element_type=jnp.float32)
    h = h + jnp.dot(xc, wc_ref[...], preferred_element_type=jnp.float32)
    h = h + jnp.dot(xn, w1n_ref[...], preferred_element_type=jnp.float32)
    h = jnp.maximum(h + b1_ref[...], 0.0)
    out_ref[...] = jnp.sum(h * w2_ref[...], axis=1) + b2_ref[0]


def _mlp(xt, xc, x_num_t, wt, wc, w1n, b1r, w2r, b2, block=4096):
    nrows = xt.shape[0]
    grid = (nrows // block,)
    wspec = lambda shape: pl.BlockSpec(shape, lambda i: (0, 0))
    return pl.pallas_call(
        _mlp_body,
        grid=grid,
        in_specs=[
            pl.BlockSpec((block, 2 * TEAM_DIM), lambda i: (i, 0)),
            pl.BlockSpec((block, 128), lambda i: (i, 0)),
            pl.BlockSpec((2, block), lambda i: (0, i)),
            wspec((2 * TEAM_DIM, HIDDEN)),
            wspec((128, HIDDEN)),
            wspec((2, HIDDEN)),
            wspec((1, HIDDEN)),
            wspec((1, HIDDEN)),
            pl.BlockSpec(memory_space=pltpu.SMEM),
        ],
        out_specs=pl.BlockSpec((block,), lambda i: (i,)),
        out_shape=jax.ShapeDtypeStruct((nrows,), jnp.float32),
    )(xt, xc, x_num_t, wt, wc, w1n, b1r, w2r, b2)


def kernel(team_a_id, team_b_id, city_id, x_num, team_emb, city_emb, W1, b1, W2, b2):
    ta = team_a_id.astype(jnp.int32)
    tb = team_b_id.astype(jnp.int32)
    tc = city_id.astype(jnp.int32)
    w1t = W1.T
    wt = w1t[0:2 * TEAM_DIM]
    wc = jnp.concatenate(
        [w1t[2 * TEAM_DIM:2 * TEAM_DIM + CITY_DIM],
         jnp.zeros((128 - CITY_DIM, HIDDEN), jnp.float32)], axis=0)
    w1n = w1t[2 * TEAM_DIM + CITY_DIM:]
    b1r = b1.reshape(1, HIDDEN)
    w2r = W2.reshape(1, HIDDEN)

    team_lin = _make_transpose()(team_emb.T).reshape(N_TEAMS, TEAM_DIM)
    half = B // 2
    gather = _make_gather(half)
    xnt = x_num.T
    outs = []
    for h in range(2):
        sl = slice(h * half, (h + 1) * half)
        xt, xc = gather(ta[sl], tb[sl], tc[sl], team_lin, city_emb)
        outs.append(_mlp(xt.reshape(half, 2 * TEAM_DIM), xc.reshape(half, 128),
                         xnt[:, sl], wt, wc, w1n, b1r, w2r, b2))
    return jnp.concatenate(outs)


# speedup vs baseline: 2.2501x; 1.3361x over previous
import functools

import jax
import jax.numpy as jnp
from jax import lax
from jax.experimental import pallas as pl
from jax.experimental.pallas import tpu as pltpu
from jax.experimental.pallas import tpu_sc as plsc

B = 16384
TEAM_DIM = 64
CITY_DIM = 32
HIDDEN = 128

NC, NS = 2, 16
NW = NC * NS
CHUNK = 128


@functools.cache
def _make_gather(nrows):
  rpw = nrows // NW
  nch = rpw // CHUNK

  def _gather_body(ta_hbm, tb_hbm, tc_hbm, team_hbm, city_hbm,
                   xt_hbm, xc_hbm,
                   idx_a, idx_b, idx_c, rows_a, rows_b, rows_c, sem):
    wid = lax.axis_index("s") * NC + lax.axis_index("c")
    base = wid * rpw
    pltpu.sync_copy(ta_hbm.at[pl.ds(base, rpw)], idx_a)
    pltpu.sync_copy(tb_hbm.at[pl.ds(base, rpw)], idx_b)
    pltpu.sync_copy(tc_hbm.at[pl.ds(base, rpw)], idx_c)
    copies = []
    for j in range(nch):
        sl = pl.ds(j * CHUNK, CHUNK)
        copies.append(pltpu.async_copy(
            team_hbm.at[idx_a.at[sl]], rows_a.at[sl], sem))
        copies.append(pltpu.async_copy(
            team_hbm.at[idx_b.at[sl]], rows_b.at[sl], sem))
        copies.append(pltpu.async_copy(
            city_hbm.at[idx_c.at[sl]], rows_c.at[sl], sem))
    for c in copies:
        c.wait()
    pltpu.sync_copy(rows_a, xt_hbm.at[wid, :, pl.ds(0, TEAM_DIM)])
    pltpu.sync_copy(rows_b, xt_hbm.at[wid, :, pl.ds(TEAM_DIM, TEAM_DIM)])
    pltpu.sync_copy(rows_c, xc_hbm.at[wid, :, pl.ds(0, CITY_DIM)])

  return pl.kernel(
    _gather_body,
    out_type=(
        jax.ShapeDtypeStruct((NW, rpw, 2 * TEAM_DIM), jnp.float32),
        jax.ShapeDtypeStruct((NW, rpw, 128), jnp.float32),
    ),
    mesh=plsc.VectorSubcoreMesh(core_axis_name="c", subcore_axis_name="s",
                                num_cores=NC, num_subcores=NS),
    scratch_types=[
        pltpu.VMEM((rpw,), jnp.int32),
        pltpu.VMEM((rpw,), jnp.int32),
        pltpu.VMEM((rpw,), jnp.int32),
        pltpu.VMEM((rpw, TEAM_DIM), jnp.float32),
        pltpu.VMEM((rpw, TEAM_DIM), jnp.float32),
        pltpu.VMEM((rpw, CITY_DIM), jnp.float32),
        pltpu.SemaphoreType.DMA,
    ],
    compiler_params=pltpu.CompilerParams(use_tc_tiling_on_sc=False),
  )


def _mlp_body(xt_ref, xc_ref, xn_ref, wt_ref, wc_ref, w1n_ref,
              b1_ref, w2_ref, b2_ref, out_ref):
    blk = xt_ref.shape[0]
    lane = lax.broadcasted_iota(jnp.int32, (blk, 128), 1)
    xc = jnp.where(lane < CITY_DIM, xc_ref[...], 0.0)
    xn = xn_ref[...].T
    h = jnp.dot(xt_ref[...], wt_ref[...], preferred_element_type=jnp.float32)
    h = h + jnp.dot(xc, wc_ref[...], preferred_element_type=jnp.float32)
    h = h + jnp.dot(xn, w1n_ref[...], preferred_element_type=jnp.float32)
    h = jnp.maximum(h + b1_ref[...], 0.0)
    out_ref[...] = jnp.sum(h * w2_ref[...], axis=1) + b2_ref[0]


def _mlp(xt, xc, x_num_t, wt, wc, w1n, b1r, w2r, b2, block=4096):
    nrows = xt.shape[0]
    grid = (nrows // block,)
    wspec = lambda shape: pl.BlockSpec(shape, lambda i: (0, 0))
    return pl.pallas_call(
        _mlp_body,
        grid=grid,
        in_specs=[
            pl.BlockSpec((block, 2 * TEAM_DIM), lambda i: (i, 0)),
            pl.BlockSpec((block, 128), lambda i: (i, 0)),
            pl.BlockSpec((2, block), lambda i: (0, i)),
            wspec((2 * TEAM_DIM, HIDDEN)),
            wspec((128, HIDDEN)),
            wspec((2, HIDDEN)),
            wspec((1, HIDDEN)),
            wspec((1, HIDDEN)),
            pl.BlockSpec(memory_space=pltpu.SMEM),
        ],
        out_specs=pl.BlockSpec((block,), lambda i: (i,)),
        out_shape=jax.ShapeDtypeStruct((nrows,), jnp.float32),
    )(xt, xc, x_num_t, wt, wc, w1n, b1r, w2r, b2)


def kernel(team_a_id, team_b_id, city_id, x_num, team_emb, city_emb, W1, b1, W2, b2):
    ta = team_a_id.astype(jnp.int32)
    tb = team_b_id.astype(jnp.int32)
    tc = city_id.astype(jnp.int32)
    w1t = W1.T
    wt = w1t[0:2 * TEAM_DIM]
    wc = jnp.concatenate(
        [w1t[2 * TEAM_DIM:2 * TEAM_DIM + CITY_DIM],
         jnp.zeros((128 - CITY_DIM, HIDDEN), jnp.float32)], axis=0)
    w1n = w1t[2 * TEAM_DIM + CITY_DIM:]
    b1r = b1.reshape(1, HIDDEN)
    w2r = W2.reshape(1, HIDDEN)

    half = B // 2
    gather = _make_gather(half)
    xnt = x_num.T
    outs = []
    for h in range(2):
        sl = slice(h * half, (h + 1) * half)
        xt, xc = gather(ta[sl], tb[sl], tc[sl], team_emb, city_emb)
        outs.append(_mlp(xt.reshape(half, 2 * TEAM_DIM), xc.reshape(half, 128),
                         xnt[:, sl], wt, wc, w1n, b1r, w2r, b2))
    return jnp.concatenate(outs)
